# baseline (speedup 1.0000x reference)
import functools

import jax
import jax.numpy as jnp
from jax.experimental import pallas as pl
from jax.experimental.pallas import tpu as pltpu


def _sigmoid(z):
    return 0.5 * jnp.tanh(0.5 * z) + 0.5


_HIDDEN_CHUNK = 4096


def _mlp_kernel(x_ref, w1_ref, w2_ref, o_ref):
    xb = x_ref[...].astype(jnp.bfloat16)
    hidden = w1_ref.shape[1]
    hc = min(_HIDDEN_CHUNK, hidden)
    acc = jnp.zeros(o_ref.shape, jnp.float32)
    for c in range(hidden // hc):
        h = jnp.dot(xb, w1_ref[:, c * hc:(c + 1) * hc],
                    preferred_element_type=jnp.float32)
        a = _sigmoid(h).astype(jnp.bfloat16)
        acc = acc + jnp.dot(a, w2_ref[c * hc:(c + 1) * hc, :],
                            preferred_element_type=jnp.float32)
    o_ref[...] = _sigmoid(acc)


@functools.partial(jax.jit, static_argnames=("batch_block",))
def _mlp_forward(x, w1, w2, batch_block=1024):
    batch, input_size = x.shape
    hidden_size, _ = w1.shape
    output_size, _ = w2.shape

    w1b = w1.T.astype(jnp.bfloat16)
    w2b = w2.T.astype(jnp.bfloat16)

    n_blocks = pl.cdiv(batch, batch_block)
    padded_batch = n_blocks * batch_block
    if padded_batch != batch:
        x = jnp.pad(x, ((0, padded_batch - batch), (0, 0)))

    out = pl.pallas_call(
        _mlp_kernel,
        out_shape=jax.ShapeDtypeStruct((padded_batch, output_size), jnp.float32),
        grid=(n_blocks,),
        in_specs=[
            pl.BlockSpec((batch_block, input_size), lambda i: (i, 0)),
            pl.BlockSpec((input_size, hidden_size), lambda i: (0, 0)),
            pl.BlockSpec((hidden_size, output_size), lambda i: (0, 0)),
        ],
        out_specs=pl.BlockSpec((batch_block, output_size), lambda i: (i, 0)),
        compiler_params=pltpu.CompilerParams(
            dimension_semantics=("parallel",),
        ),
    )(x, w1b, w2b)

    if padded_batch != batch:
        out = out[:batch]
    return out


def kernel(x, w1, w2):
    return _mlp_forward(x, w1, w2)


# speedup vs baseline: 1.0380x; 1.0380x over previous
import functools

import jax
import jax.numpy as jnp
from jax.experimental import pallas as pl
from jax.experimental.pallas import tpu as pltpu


def _mlp_kernel(x_ref, w1_ref, w2_ref, b2_ref, o_ref):
    xb = x_ref[...].astype(jnp.bfloat16)
    t = jnp.tanh(jax.lax.dot_general(
        xb, w1_ref[...], (((1,), (1,)), ((), ())),
        preferred_element_type=jnp.float32)).astype(jnp.bfloat16)
    y = jax.lax.dot_general(
        t, w2_ref[...], (((1,), (1,)), ((), ())),
        preferred_element_type=jnp.float32)
    o_ref[...] = 0.5 * jnp.tanh(y + b2_ref[...]) + 0.5


@functools.partial(jax.jit, static_argnames=("batch_block",))
def _mlp_forward(x, w1, w2, batch_block=512):
    batch, input_size = x.shape
    hidden_size, _ = w1.shape
    output_size, _ = w2.shape

    w1b = (0.5 * w1).astype(jnp.bfloat16)
    w2b = (0.25 * w2).astype(jnp.bfloat16)
    b2 = (0.25 * jnp.sum(w2, axis=1, dtype=jnp.float32)).reshape(1, output_size)

    n_blocks = pl.cdiv(batch, batch_block)
    padded_batch = n_blocks * batch_block
    if padded_batch != batch:
        x = jnp.pad(x, ((0, padded_batch - batch), (0, 0)))

    out = pl.pallas_call(
        _mlp_kernel,
        out_shape=jax.ShapeDtypeStruct((padded_batch, output_size), jnp.float32),
        grid=(n_blocks,),
        in_specs=[
            pl.BlockSpec((batch_block, input_size), lambda i: (i, 0)),
            pl.BlockSpec((hidden_size, input_size), lambda i: (0, 0)),
            pl.BlockSpec((output_size, hidden_size), lambda i: (0, 0)),
            pl.BlockSpec((1, output_size), lambda i: (0, 0)),
        ],
        out_specs=pl.BlockSpec((batch_block, output_size), lambda i: (i, 0)),
        compiler_params=pltpu.CompilerParams(
            dimension_semantics=("parallel",),
        ),
    )(x, w1b, w2b, b2)

    if padded_batch != batch:
        out = out[:batch]
    return out


def kernel(x, w1, w2):
    return _mlp_forward(x, w1, w2)
